# Initial kernel scaffold; baseline (speedup 1.0000x reference)
import jax
import jax.numpy as jnp
from jax.experimental import pallas as pl


def kernel(x, enc_Ws, enc_bs, dec_Ws, dec_bs, codebooks, temperature):
    raise NotImplementedError("write your pallas kernel here")


# speedup vs baseline: 2.1674x; 2.1674x over previous
import jax
import jax.numpy as jnp
from jax.experimental import pallas as pl
from jax.experimental.pallas import tpu as pltpu

B = 4096
BS = 512
NB = B // BS
NC = 3
K = 1024
D = 256
BETA = 0.25


def _rq_body(x_ref,
             ew0, ew1, ew2, ew3, eb0, eb1, eb2, eb3,
             dw0, dw1, dw2, dw3, db0, db1, db2, db3,
             cb_ref,
             norms_ref, keys_ref, stats_ref):
    i = pl.program_id(0)
    xb = x_ref[...]

    def mm(a, w):
        return jax.lax.dot_general(
            a, w, (((1,), (0,)), ((), ())),
            preferred_element_type=jnp.float32)

    h = mm(xb, ew0[...]) + eb0[...]
    h = jnp.maximum(h, 0.0)
    h = mm(h, ew1[...]) + eb1[...]
    h = jnp.maximum(h, 0.0)
    h = mm(h, ew2[...]) + eb2[...]
    h = jnp.maximum(h, 0.0)
    res = mm(h, ew3[...]) + eb3[...]

    jidx = jax.lax.broadcasted_iota(jnp.int32, (BS, K), 1)
    ones_row = jnp.ones((1, D), jnp.float32)

    qsum = jnp.zeros((BS, 1), jnp.float32)
    z_hat = jnp.zeros((BS, D), jnp.float32)
    key_col = jnp.zeros((BS, 1), jnp.int32)
    norm_cols = []
    for c in range(NC):
        cb = cb_ref[c]
        cn_row = jax.lax.dot_general(
            ones_row, cb * cb, (((1,), (1,)), ((), ())),
            preferred_element_type=jnp.float32)
        rn_col = jnp.sum(res * res, axis=1, keepdims=True)
        dot = jax.lax.dot_general(
            res, cb, (((1,), (1,)), ((), ())),
            preferred_element_type=jnp.float32)
        d = rn_col - 2.0 * dot + cn_row
        dmin = jnp.min(d, axis=1, keepdims=True)
        ids_col = jnp.min(jnp.where(d == dmin, jidx, K), axis=1,
                          keepdims=True)
        onehot = (jidx == ids_col).astype(jnp.float32)
        emb = mm(onehot, cb)
        diff = res - emb
        qsum = qsum + (BETA + 1.0) * jnp.sum(diff * diff, axis=1,
                                             keepdims=True)
        emb_st = res + (emb - res)
        norm_cols.append(jnp.sqrt(jnp.sum(emb_st * emb_st, axis=1,
                                          keepdims=True)))
        z_hat = z_hat + emb_st
        key_col = key_col + ids_col * (K ** c)
        res = res - emb_st

    norms_ref[0] = jnp.concatenate(norm_cols, axis=1)
    keys_ref[0] = key_col

    h = mm(z_hat, dw0[...]) + db0[...]
    h = jnp.maximum(h, 0.0)
    h = mm(h, dw1[...]) + db1[...]
    h = jnp.maximum(h, 0.0)
    h = mm(h, dw2[...]) + db2[...]
    h = jnp.maximum(h, 0.0)
    x_hat = mm(h, dw3[...]) + db3[...]

    r = x_hat - xb
    recon_blk = jnp.sum(r * r)
    qloss_blk = jnp.sum(qsum)

    @pl.when(i == 0)
    def _init():
        stats_ref[3:4, :] = jnp.full((1, 128), recon_blk, jnp.float32)
        stats_ref[4:5, :] = jnp.full((1, 128), qloss_blk, jnp.float32)

    @pl.when(i > 0)
    def _acc():
        stats_ref[3:4, :] = stats_ref[3:4, :] + recon_blk
        stats_ref[4:5, :] = stats_ref[4:5, :] + qloss_blk

    @pl.when(i == NB - 1)
    def _final():
        rs = stats_ref[3, 0]
        qs = stats_ref[4, 0]
        stats_ref[0:1, :] = jnp.full((1, 128), (rs + qs) / B, jnp.float32)
        stats_ref[1:2, :] = jnp.full((1, 128), rs / B, jnp.float32)
        stats_ref[2:3, :] = jnp.full((1, 128), qs / B, jnp.float32)


def _unique_body(krow_ref, kcol_ref, out_ref):
    krow = krow_ref[...]
    CH = 512
    cnt = jnp.zeros((), jnp.int32)
    for ci in range(B // CH):
        a = kcol_ref[pl.ds(ci * CH, CH), :]
        eq = a == krow
        jidx = jax.lax.broadcasted_iota(jnp.int32, (CH, B), 1)
        iidx = jax.lax.broadcasted_iota(jnp.int32, (CH, B), 0) + ci * CH
        dup = jnp.where(eq & (jidx > iidx), 1, 0)
        has = jnp.max(dup, axis=1)
        cnt = cnt + jnp.sum(has)
    p = (B - cnt).astype(jnp.float32) / B
    out_ref[...] = jnp.full((1, 128), p, jnp.float32)


def kernel(x, enc_Ws, enc_bs, dec_Ws, dec_bs, codebooks, temperature):
    del temperature
    enc_bs = [b.reshape(1, -1) for b in enc_bs]
    dec_bs = [b.reshape(1, -1) for b in dec_bs]

    full = lambda a: pl.BlockSpec(a.shape, lambda i: (0,) * a.ndim)
    in_specs = [pl.BlockSpec((BS, 768), lambda i: (i, 0))]
    in_specs += [full(w) for w in enc_Ws] + [full(b) for b in enc_bs]
    in_specs += [full(w) for w in dec_Ws] + [full(b) for b in dec_bs]
    in_specs += [full(codebooks)]

    norms, keys, stats = pl.pallas_call(
        _rq_body,
        grid=(NB,),
        in_specs=in_specs,
        out_specs=[
            pl.BlockSpec((1, BS, NC), lambda i: (i, 0, 0)),
            pl.BlockSpec((1, BS, 1), lambda i: (i, 0, 0)),
            pl.BlockSpec((8, 128), lambda i: (0, 0)),
        ],
        out_shape=[
            jax.ShapeDtypeStruct((NB, BS, NC), jnp.float32),
            jax.ShapeDtypeStruct((NB, BS, 1), jnp.int32),
            jax.ShapeDtypeStruct((8, 128), jnp.float32),
        ],
        compiler_params=pltpu.CompilerParams(
            dimension_semantics=("arbitrary",)),
    )(x, *enc_Ws, *enc_bs, *dec_Ws, *dec_bs, codebooks)

    kcol = keys.reshape(B, 1)
    krow = keys.reshape(1, B)
    p_unique = pl.pallas_call(
        _unique_body,
        in_specs=[pl.BlockSpec((1, B), lambda: (0, 0)),
                  pl.BlockSpec((B, 1), lambda: (0, 0))],
        out_specs=pl.BlockSpec((1, 128), lambda: (0, 0)),
        out_shape=jax.ShapeDtypeStruct((1, 128), jnp.float32),
    )(krow, kcol)

    embs_norm = norms.reshape(B, NC)
    loss = stats[0, 0]
    mean_recon = stats[1, 0]
    mean_qloss = stats[2, 0]
    return (loss, mean_recon, mean_qloss, embs_norm, p_unique[0, 0])
